# Initial kernel scaffold; baseline (speedup 1.0000x reference)
import jax
import jax.numpy as jnp
from jax.experimental import pallas as pl


def kernel(x, edge_index, edge_attr, batch, emb, W1, b1, W2, b2, p1, p2, gW, gb, pW1, pb1, pW2, pb2):
    raise NotImplementedError("write your pallas kernel here")


# speedup vs baseline: 5.2223x; 5.2223x over previous
import functools

import jax
import jax.numpy as jnp
from jax import lax
from jax.experimental import pallas as pl
from jax.experimental.pallas import tpu as pltpu
from jax.experimental.pallas import tpu_sc as plsc

NC, NS, LANES = 2, 16, 16
NW = NC * NS

_SC_MESH = plsc.VectorSubcoreMesh(core_axis_name="c", subcore_axis_name="s")
_SC_PARAMS = pltpu.CompilerParams(needs_layout_passes=False)


def _f32(shape):
    return jax.ShapeDtypeStruct(shape, jnp.float32)


def _sc_embed(N, L, H, V):
    C = 40
    n_chunks = N // C
    iters = (n_chunks + NW - 1) // NW

    @functools.partial(
        pl.kernel,
        out_type=_f32((N, H)),
        mesh=_SC_MESH,
        compiler_params=_SC_PARAMS,
        scratch_types=[
            pltpu.VMEM((C * L,), jnp.int32),
            pltpu.VMEM((C * L, H), jnp.float32),
            pltpu.VMEM((C, H), jnp.float32),
            pltpu.VMEM((1, H), jnp.float32),
            pltpu.SemaphoreType.DMA,
        ],
    )
    def k(x_hbm, emb_hbm, out_hbm, xi_v, tok_v, hv, emb0_v, sem):
        ci = lax.axis_index("c")
        si = lax.axis_index("s")
        w = ci * NS + si
        pltpu.sync_copy(emb_hbm.at[pl.ds(0, 1)], emb0_v)

        def chunk_body(it, _):
            chunk = w + it * NW

            @pl.when(chunk < n_chunks)
            def _():
                pltpu.sync_copy(x_hbm.at[pl.ds(chunk * C * L, C * L)], xi_v)
                pltpu.async_copy(emb_hbm.at[xi_v], tok_v, sem).wait()

                def node_body(n, _):
                    toks = xi_v[pl.ds(n * L, L)]
                    cnt = plsc.all_reduce_population_count(toks != 0)
                    cntf = cnt.astype(jnp.float32)
                    nzero = jnp.float32(L) - cntf
                    recip = 1.0 / jnp.maximum(cntf, 1.0)
                    for j in range(H // LANES):
                        sl = pl.ds(j * LANES, LANES)
                        acc = jnp.zeros((LANES,), jnp.float32)
                        for t in range(L):
                            acc = acc + tok_v[n * L + t, sl]
                        hv[n, sl] = (acc - nzero * emb0_v[0, sl]) * recip
                    return 0

                lax.fori_loop(0, C, node_body, 0)
                pltpu.sync_copy(hv, out_hbm.at[pl.ds(chunk * C, C)])
            return 0

        lax.fori_loop(0, iters, chunk_body, 0)

    return k


def _sc_deg(NP, NCH, K):
    G = K // LANES
    NR = NP // 128

    @functools.partial(
        pl.kernel,
        out_type=_f32((NW, NR, 128)),
        mesh=_SC_MESH,
        compiler_params=_SC_PARAMS,
        scratch_types=[
            pltpu.VMEM((NCH, K), jnp.int32),
            pltpu.VMEM((NCH, K), jnp.int32),
            pltpu.VMEM((NCH, K), jnp.float32),
            pltpu.VMEM((NR, 128), jnp.float32),
            pltpu.VMEM((NR, 128), jnp.float32),
        ],
    )
    def k(row_hbm, col_hbm, ew_hbm, u_hbm, out_hbm, rc, cc, wc, uv, acc):
        ci = lax.axis_index("c")
        si = lax.axis_index("s")
        w = ci * NS + si
        pltpu.sync_copy(u_hbm, uv)
        pltpu.sync_copy(row_hbm.at[w], rc)
        pltpu.sync_copy(col_hbm.at[w], cc)
        pltpu.sync_copy(ew_hbm.at[w], wc)

        def zero_body(i, _):
            for j in range(128 // LANES):
                acc[i, pl.ds(j * LANES, LANES)] = jnp.zeros((LANES,), jnp.float32)
            return 0

        lax.fori_loop(0, NR, zero_body, 0)

        def edge_body(i, _):
            for g in range(G):
                sl = pl.ds(g * LANES, LANES)
                r16 = rc[i, sl]
                c16 = cc[i, sl]
                w16 = wc[i, sl]
                ur = plsc.load_gather(uv, [r16 >> 7, r16 & 127])
                plsc.addupdate_scatter(acc, [c16 >> 7, c16 & 127], w16 * ur)
            return 0

        lax.fori_loop(0, NCH, edge_body, 0)
        pltpu.sync_copy(acc, out_hbm.at[w])

    return k


def _sc_msg(NP, H, NG, CG, K):
    G = K // LANES
    RPS = NP // NS
    NB = RPS // K

    @functools.partial(
        pl.kernel,
        out_type=_f32((NC, NP, H)),
        mesh=_SC_MESH,
        compiler_params=_SC_PARAMS,
        scratch_types=[
            pltpu.VMEM((CG, K), jnp.int32),
            pltpu.VMEM((CG, K), jnp.int32),
            pltpu.VMEM((CG, K), jnp.float32),
            pltpu.VMEM((K, H), jnp.float32),
            pltpu.VMEM_SHARED((NP, H), jnp.float32),
            pltpu.SemaphoreType.DMA,
        ],
    )
    def k(row_hbm, col_hbm, ew_hbm, y_hbm, out_hbm, rc, cc, wc, rows, accS, sem):
        ci = lax.axis_index("c")
        si = lax.axis_index("s")
        w = ci * NS + si

        def zb_body(i, _):
            for j in range(H // LANES):
                rows[i, pl.ds(j * LANES, LANES)] = jnp.zeros((LANES,), jnp.float32)
            return 0

        lax.fori_loop(0, K, zb_body, 0)
        for t in range(NB):
            pltpu.sync_copy(rows, accS.at[pl.ds(si * RPS + t * K, K)])
        plsc.subcore_barrier()

        eids0 = lax.iota(jnp.int32, LANES)

        def group_body(gr, _):
            pltpu.sync_copy(row_hbm.at[w, gr], rc)
            pltpu.sync_copy(col_hbm.at[w, gr], cc)
            pltpu.sync_copy(ew_hbm.at[w, gr], wc)

            def chunk_body(j, _):
                pltpu.async_copy(y_hbm.at[rc.at[j]], rows, sem).wait()
                jv = jnp.full((LANES,), j, jnp.int32)
                for g in range(G):
                    eids = eids0 + g * LANES
                    cf = plsc.load_gather(wc, [jv, eids])
                    for d in range(H):
                        dv = jnp.full((LANES,), d, jnp.int32)
                        v = plsc.load_gather(rows, [eids, dv]) * cf
                        plsc.store_scatter(rows, [eids, dv], v)
                pltpu.sync_copy(rows, accS.at[cc.at[j]], add=True)
                return 0

            lax.fori_loop(0, CG, chunk_body, 0)
            return 0

        lax.fori_loop(0, NG, group_body, 0)
        plsc.subcore_barrier()
        for t in range(NB):
            sl = pl.ds(si * RPS + t * K, K)
            pltpu.sync_copy(accS.at[sl], rows)
            pltpu.sync_copy(rows, out_hbm.at[ci, sl])

    return k


def _tc_ew(E):
    def body(eaT_ref, out_ref):
        out_ref[...] = (eaT_ref[0] + eaT_ref[1] + eaT_ref[2] + eaT_ref[3]) * 0.25

    return pl.pallas_call(body, out_shape=_f32((E,)))


def _tc_mid1(N, H):
    def body(degpT_ref, h0_ref, W1_ref, dis_ref, y_ref):
        deg = jnp.sum(degpT_ref[...], axis=1, keepdims=True) + 1.0
        dis = lax.rsqrt(jnp.maximum(deg, 1e-30))
        dis_ref[...] = dis
        y_ref[...] = jnp.dot(h0_ref[...], W1_ref[...],
                             preferred_element_type=jnp.float32) * dis

    return pl.pallas_call(body, out_shape=[_f32((N, 1)), _f32((N, H))])


def _tc_mid2(N, H):
    def body(degpT_ref, u_ref, xw_ref, dis_ref, y_ref):
        u = u_ref[...]
        deg = u * (jnp.sum(degpT_ref[...], axis=1, keepdims=True) + 1.0)
        pos = deg > 0
        dis = jnp.where(pos, lax.rsqrt(jnp.where(pos, deg, 1.0)), 0.0)
        dis_ref[...] = dis
        y_ref[...] = xw_ref[...] * dis

    return pl.pallas_call(body, out_shape=[_f32((N, 1)), _f32((N, H))])


def _topk_mask(score, k):
    n = score.shape[0]
    ku = lax.bitcast_convert_type(score, jnp.uint32)
    key = jnp.where((ku >> 31) == 1, ~ku, ku | jnp.uint32(0x80000000))

    def t_body(i, T):
        cand = T | (jnp.uint32(1) << (jnp.uint32(31) - i.astype(jnp.uint32)))
        cnt = jnp.sum((key >= cand).astype(jnp.int32))
        return jnp.where(cnt >= k, cand, T)

    T = lax.fori_loop(0, 32, t_body, jnp.uint32(0))
    count_gt = jnp.sum((key > T).astype(jnp.int32))
    needed = k - count_gt
    idx = lax.broadcasted_iota(jnp.int32, (n, 1), 0)
    eq = key == T

    def l_body(i, L):
        cand = L | (jnp.int32(1) << (jnp.int32(13) - i))
        g = jnp.sum((eq & (idx < cand)).astype(jnp.int32))
        return jnp.where(g < needed, cand, L)

    L = lax.fori_loop(0, 14, l_body, jnp.int32(0))
    return (key > T) | (eq & (idx <= L))


def _attpool(xp, sel, gW, gb):
    g = jnp.dot(xp, gW, preferred_element_type=jnp.float32) + gb
    m = jnp.max(jnp.where(sel, g, -jnp.inf))
    e = jnp.where(sel, jnp.exp(g - m), 0.0)
    aw = e / jnp.sum(e)
    return jnp.sum(xp * aw, axis=0, keepdims=True)


def _tc_big1(N, H, k1):
    def body(part_ref, y_ref, dis_ref, b1_ref, p1_ref, gW_ref, gb_ref, W2_ref,
             u_ref, xw2_ref, out1_ref):
        y = y_ref[...]
        h = jnp.maximum(dis_ref[...] * (part_ref[0] + part_ref[1] + y) + b1_ref[...], 0.0)
        p1 = p1_ref[...]
        score = jnp.dot(h, p1, preferred_element_type=jnp.float32) / jnp.sqrt(jnp.sum(p1 * p1))
        sel = _topk_mask(score, k1)
        self_f = sel.astype(jnp.float32)
        xp = h * (jnp.tanh(score) * self_f)
        u_ref[...] = self_f
        out1_ref[...] = _attpool(xp, sel, gW_ref[...], gb_ref[0, 0])
        xw2_ref[...] = jnp.dot(xp, W2_ref[...], preferred_element_type=jnp.float32)

    return pl.pallas_call(
        body, out_shape=[_f32((N, 1)), _f32((N, H)), _f32((1, H))],
        compiler_params=pltpu.CompilerParams(vmem_limit_bytes=100 * 1024 * 1024))


def _tc_big2(N, H, k2):
    def body(part_ref, y_ref, dis_ref, u_ref, b2_ref, p2_ref, gW_ref, gb_ref,
             out1_ref, pW1_ref, pb1_ref, pW2_ref, pb2_ref,
             logits_ref, act_ref):
        u = u_ref[...]
        y = y_ref[...]
        h2 = jnp.maximum(dis_ref[...] * (part_ref[0] + part_ref[1] + y) + b2_ref[...], 0.0) * u
        p2 = p2_ref[...]
        score2 = jnp.dot(h2, p2, preferred_element_type=jnp.float32) / jnp.sqrt(jnp.sum(p2 * p2))
        score2m = jnp.where(u > 0, score2, -jnp.inf)
        sel2 = _topk_mask(score2m, k2)
        xp2 = h2 * (jnp.tanh(score2m) * sel2.astype(jnp.float32))
        out2 = _attpool(xp2, sel2, gW_ref[...], gb_ref[0, 0])
        act = out1_ref[...] + out2
        act_ref[...] = act
        hid = jnp.maximum(jnp.dot(act, pW1_ref[...], preferred_element_type=jnp.float32)
                          + pb1_ref[...], 0.0)
        z = jnp.dot(hid, pW2_ref[...], preferred_element_type=jnp.float32) + pb2_ref[...]
        logits_ref[...] = z * lax.rsqrt(jnp.sum(z * z))

    return pl.pallas_call(
        body, out_shape=[_f32((1, H)), _f32((1, H))],
        compiler_params=pltpu.CompilerParams(vmem_limit_bytes=100 * 1024 * 1024))


def kernel(x, edge_index, edge_attr, batch, emb, W1, b1, W2, b2, p1, p2, gW, gb,
           pW1, pb1, pW2, pb2):
    N, L = x.shape
    E = edge_index.shape[1]
    V, H = emb.shape
    K = 80
    EPW = E // NW
    NCH = EPW // K
    NP = ((N + NS * K - 1) // (NS * K)) * (NS * K) if (NS * K) % 128 == 0 else ((N + 1279) // 1280) * 1280
    NG, CG = 5, NCH // 5
    k1 = (N + 1) // 2
    k2 = (k1 + 1) // 2

    row3 = edge_index[0].reshape(NW, NCH, K)
    col3 = edge_index[1].reshape(NW, NCH, K)
    ew = _tc_ew(E)(edge_attr.T)
    ew3 = ew.reshape(NW, NCH, K)
    row4 = row3.reshape(NW, NG, CG, K)
    col4 = col3.reshape(NW, NG, CG, K)
    ew4 = ew3.reshape(NW, NG, CG, K)
    h0 = _sc_embed(N, L, H, V)(x.reshape(-1), emb)
    ones = jnp.ones((N,), jnp.float32)
    up1 = jnp.zeros((NP,), jnp.float32).at[:N].set(ones).reshape(NP // 128, 128)
    degp1 = _sc_deg(NP, NCH, K)(row3, col3, ew3, up1)
    degpT1 = degp1.reshape(NW, NP)[:, :N].T
    dis1, y1 = _tc_mid1(N, H)(degpT1, h0, W1)
    part1 = _sc_msg(NP, H, NG, CG, K)(row4, col4, ew4, y1)[:, :N, :]
    u, xw2, out1 = _tc_big1(N, H, k1)(
        part1, y1, dis1, b1.reshape(1, H), p1.reshape(H, 1), gW,
        gb.reshape(1, 1), W2)
    up2 = jnp.zeros((NP,), jnp.float32).at[:N].set(u.reshape(-1)).reshape(NP // 128, 128)
    degp2 = _sc_deg(NP, NCH, K)(row3, col3, ew3, up2)
    degpT2 = degp2.reshape(NW, NP)[:, :N].T
    dis2, y2 = _tc_mid2(N, H)(degpT2, u, xw2)
    part2 = _sc_msg(NP, H, NG, CG, K)(row4, col4, ew4, y2)[:, :N, :]
    logits, act = _tc_big2(N, H, k2)(
        part2, y2, dis2, u, b2.reshape(1, H), p2.reshape(H, 1), gW,
        gb.reshape(1, 1), out1, pW1, pb1.reshape(1, H), pW2, pb2.reshape(1, H))
    return (logits, act)
